# Initial kernel scaffold; baseline (speedup 1.0000x reference)
import jax
import jax.numpy as jnp
from jax.experimental import pallas as pl


def kernel(pred, gt):
    raise NotImplementedError("write your pallas kernel here")


# speedup vs baseline: 93.9872x; 93.9872x over previous
import jax
import jax.numpy as jnp
from jax.experimental import pallas as pl
from jax.experimental.pallas import tpu as pltpu

_K = 10
_N = 2048
_B = 8
_R = 256
_L = 128
_INTERPRET = False


def _normals_body(prow_ref, pall_ref, pallT_ref, out_ref):
    prow = prow_ref[0]
    pall = pall_ref[0]
    pallT = pallT_ref[0]

    g = jax.lax.dot_general(prow, pallT, (((1,), (0,)), ((), ())),
                            preferred_element_type=jnp.float32)
    sq_row = jnp.sum(prow * prow, axis=1, keepdims=True)
    sq_all = jnp.sum(pallT * pallT, axis=0, keepdims=True)
    d = sq_row - 2.0 * g + sq_all

    work = d
    t = None
    for _ in range(_K):
        t = jnp.min(work, axis=1, keepdims=True)
        work = jnp.where(work <= t, jnp.inf, work)
    mask = (d <= t).astype(jnp.float32)

    x = pall[:, 0:1]
    y = pall[:, 1:2]
    z = pall[:, 2:3]
    feats = jnp.concatenate(
        [x, y, z, x * x, y * y, z * z, x * y, x * z, y * z,
         jnp.zeros((_N, _L - 9), jnp.float32)], axis=1)
    s = jax.lax.dot_general(mask, feats, (((1,), (0,)), ((), ())),
                            preferred_element_type=jnp.float32)

    inv_k = 1.0 / _K
    mx = s[:, 0:1] * inv_k
    my = s[:, 1:2] * inv_k
    mz = s[:, 2:3] * inv_k
    cxx = s[:, 3:4] * inv_k - mx * mx
    cyy = s[:, 4:5] * inv_k - my * my
    czz = s[:, 5:6] * inv_k - mz * mz
    cxy = s[:, 6:7] * inv_k - mx * my
    cxz = s[:, 7:8] * inv_k - mx * mz
    cyz = s[:, 8:9] * inv_k - my * mz

    q = (cxx + cyy + czz) * (1.0 / 3.0)
    aa = cxx - q
    bb = cyy - q
    cc = czz - q
    p2 = aa * aa + bb * bb + cc * cc + 2.0 * (cxy * cxy + cxz * cxz + cyz * cyz)
    p = jnp.sqrt(p2 * (1.0 / 6.0) + 1e-38)
    pinv = 1.0 / p
    b11 = aa * pinv
    b22 = bb * pinv
    b33 = cc * pinv
    b12 = cxy * pinv
    b13 = cxz * pinv
    b23 = cyz * pinv
    detb = (b11 * (b22 * b33 - b23 * b23)
            - b12 * (b12 * b33 - b23 * b13)
            + b13 * (b12 * b23 - b22 * b13))
    r = jnp.clip(detb * 0.5, -1.0, 1.0)
    lam = jnp.full_like(r, -2.0)
    for _ in range(12):
        f = lam * lam * lam - 3.0 * lam - 2.0 * r
        fp = 3.0 * lam * lam - 3.0 + 1e-10
        lam = lam - f / fp
    lmin = q + p * lam

    m11 = cxx - lmin
    m22 = cyy - lmin
    m33 = czz - lmin
    c12x = cxy * cyz - cxz * m22
    c12y = cxz * cxy - m11 * cyz
    c12z = m11 * m22 - cxy * cxy
    c13x = cxy * m33 - cxz * cyz
    c13y = cxz * cxz - m11 * m33
    c13z = m11 * cyz - cxy * cxz
    c23x = m22 * m33 - cyz * cyz
    c23y = cyz * cxz - cxy * m33
    c23z = cxy * cyz - m22 * cxz
    n12 = c12x * c12x + c12y * c12y + c12z * c12z
    n13 = c13x * c13x + c13y * c13y + c13z * c13z
    n23 = c23x * c23x + c23y * c23y + c23z * c23z
    use12 = jnp.logical_and(n12 >= n13, n12 >= n23)
    use13 = jnp.logical_and(jnp.logical_not(use12), n13 >= n23)
    vx = jnp.where(use12, c12x, jnp.where(use13, c13x, c23x))
    vy = jnp.where(use12, c12y, jnp.where(use13, c13y, c23y))
    vz = jnp.where(use12, c12z, jnp.where(use13, c13z, c23z))
    inv = jax.lax.rsqrt(vx * vx + vy * vy + vz * vz + 1e-38)
    out_ref[0] = jnp.concatenate(
        [vx * inv, vy * inv, vz * inv,
         jnp.zeros((_R, _L - 3), jnp.float32)], axis=1)


def _loss_body(n_ref, out_ref):
    nall = n_ref[...]
    prod = nall[0:4] * nall[4:8]
    cos = jnp.sum(prod[:, :, 0:3], axis=-1)
    out_ref[0, 0] = 1.0 - jnp.mean(jnp.abs(cos))


def kernel(pred, gt):
    pts = jnp.concatenate([pred, gt], axis=0)
    p = jnp.transpose(pts, (0, 2, 1))
    ppad = jnp.pad(p, ((0, 0), (0, 0), (0, _L - 3)))
    ppadT = jnp.transpose(ppad, (0, 2, 1))

    normals = pl.pallas_call(
        _normals_body,
        grid=(_B, _N // _R),
        in_specs=[
            pl.BlockSpec((1, _R, _L), lambda c, rb: (c, rb, 0)),
            pl.BlockSpec((1, _N, _L), lambda c, rb: (c, 0, 0)),
            pl.BlockSpec((1, _L, _N), lambda c, rb: (c, 0, 0)),
        ],
        out_specs=pl.BlockSpec((1, _R, _L), lambda c, rb: (c, rb, 0)),
        out_shape=jax.ShapeDtypeStruct((_B, _N, _L), jnp.float32),
        interpret=_INTERPRET,
    )(ppad, ppad, ppadT)

    loss = pl.pallas_call(
        _loss_body,
        grid=(1,),
        in_specs=[pl.BlockSpec((_B, _N, _L), lambda i: (0, 0, 0))],
        out_specs=pl.BlockSpec(memory_space=pltpu.SMEM),
        out_shape=jax.ShapeDtypeStruct((1, 1), jnp.float32),
        interpret=_INTERPRET,
    )(normals)
    return loss[0, 0]


# speedup vs baseline: 126.0963x; 1.3416x over previous
import jax
import jax.numpy as jnp
from jax.experimental import pallas as pl
from jax.experimental.pallas import tpu as pltpu

_K = 10
_N = 2048
_R = 256
_NB = _N // _R
_L = 128
_INTERPRET = False


def _build_feats(pall, fhi_ref, flo_ref):
    x = pall[:, 0:1]
    y = pall[:, 1:2]
    z = pall[:, 2:3]
    f = jnp.concatenate(
        [x, y, z, x * x, y * y, z * z, x * y, x * z, y * z,
         jnp.zeros((_N, _L - 9), jnp.float32)], axis=1)
    hi = f.astype(jnp.bfloat16)
    fhi_ref[...] = hi
    flo_ref[...] = (f - hi.astype(jnp.float32)).astype(jnp.bfloat16)


def _normals(prow, pallT, fhi_ref, flo_ref):
    g = jax.lax.dot_general(prow, pallT, (((1,), (0,)), ((), ())),
                            preferred_element_type=jnp.float32)
    sq_row = jnp.sum(prow * prow, axis=1, keepdims=True)
    sq_all = jnp.sum(pallT * pallT, axis=0, keepdims=True)
    d = sq_row - 2.0 * g + sq_all

    t = jnp.min(d, axis=1, keepdims=True)
    for _ in range(_K - 1):
        t = jnp.min(jnp.where(d > t, d, jnp.inf), axis=1, keepdims=True)
    mask = (d <= t).astype(jnp.bfloat16)

    dims = (((1,), (0,)), ((), ()))
    s = (jax.lax.dot_general(mask, fhi_ref[...], dims,
                             preferred_element_type=jnp.float32)
         + jax.lax.dot_general(mask, flo_ref[...], dims,
                               preferred_element_type=jnp.float32))

    inv_k = 1.0 / _K
    mx = s[:, 0:1] * inv_k
    my = s[:, 1:2] * inv_k
    mz = s[:, 2:3] * inv_k
    cxx = s[:, 3:4] * inv_k - mx * mx
    cyy = s[:, 4:5] * inv_k - my * my
    czz = s[:, 5:6] * inv_k - mz * mz
    cxy = s[:, 6:7] * inv_k - mx * my
    cxz = s[:, 7:8] * inv_k - mx * mz
    cyz = s[:, 8:9] * inv_k - my * mz

    q = (cxx + cyy + czz) * (1.0 / 3.0)
    aa = cxx - q
    bb = cyy - q
    cc = czz - q
    p2 = aa * aa + bb * bb + cc * cc + 2.0 * (cxy * cxy + cxz * cxz + cyz * cyz)
    p = jnp.sqrt(p2 * (1.0 / 6.0) + 1e-38)
    pinv = 1.0 / p
    b11 = aa * pinv
    b22 = bb * pinv
    b33 = cc * pinv
    b12 = cxy * pinv
    b13 = cxz * pinv
    b23 = cyz * pinv
    detb = (b11 * (b22 * b33 - b23 * b23)
            - b12 * (b12 * b33 - b23 * b13)
            + b13 * (b12 * b23 - b22 * b13))
    r = jnp.clip(detb * 0.5, -1.0, 1.0)
    lam = jnp.full_like(r, -2.0)
    for _ in range(12):
        f = lam * lam * lam - 3.0 * lam - 2.0 * r
        fp = 3.0 * lam * lam - 3.0 + 1e-10
        lam = lam - f / fp
    lmin = q + p * lam

    m11 = cxx - lmin
    m22 = cyy - lmin
    m33 = czz - lmin
    c12x = cxy * cyz - cxz * m22
    c12y = cxz * cxy - m11 * cyz
    c12z = m11 * m22 - cxy * cxy
    c13x = cxy * m33 - cxz * cyz
    c13y = cxz * cxz - m11 * m33
    c13z = m11 * cyz - cxy * cxz
    c23x = m22 * m33 - cyz * cyz
    c23y = cyz * cxz - cxy * m33
    c23z = cxy * cyz - m22 * cxz
    n12 = c12x * c12x + c12y * c12y + c12z * c12z
    n13 = c13x * c13x + c13y * c13y + c13z * c13z
    n23 = c23x * c23x + c23y * c23y + c23z * c23z
    use12 = jnp.logical_and(n12 >= n13, n12 >= n23)
    use13 = jnp.logical_and(jnp.logical_not(use12), n13 >= n23)
    vx = jnp.where(use12, c12x, jnp.where(use13, c13x, c23x))
    vy = jnp.where(use12, c12y, jnp.where(use13, c13y, c23y))
    vz = jnp.where(use12, c12z, jnp.where(use13, c13z, c23z))
    inv = jax.lax.rsqrt(vx * vx + vy * vy + vz * vz + 1e-38)
    return vx * inv, vy * inv, vz * inv


def _body(prow_p_ref, pall_p_ref, pallT_p_ref,
          prow_g_ref, pall_g_ref, pallT_g_ref,
          out_ref, fphi, fplo, fghi, fglo, acc):
    rb = pl.program_id(1)

    @pl.when(rb == 0)
    def _():
        _build_feats(pall_p_ref[0], fphi, fplo)
        _build_feats(pall_g_ref[0], fghi, fglo)
        acc[0, 0] = 0.0

    pnx, pny, pnz = _normals(prow_p_ref[0], pallT_p_ref[0], fphi, fplo)
    gnx, gny, gnz = _normals(prow_g_ref[0], pallT_g_ref[0], fghi, fglo)
    cos = pnx * gnx + pny * gny + pnz * gnz
    acc[0, 0] += jnp.sum(1.0 - jnp.abs(cos))

    @pl.when(rb == _NB - 1)
    def _():
        out_ref[...] = jnp.full((1, 1, _L), acc[0, 0], jnp.float32)


def kernel(pred, gt):
    pts = jnp.concatenate([pred, gt], axis=0)
    p = jnp.transpose(pts, (0, 2, 1))
    ppad = jnp.pad(p, ((0, 0), (0, 0), (0, _L - 3)))
    ppadT = jnp.transpose(ppad, (0, 2, 1))

    partials = pl.pallas_call(
        _body,
        grid=(4, _NB),
        in_specs=[
            pl.BlockSpec((1, _R, _L), lambda c, rb: (c, rb, 0)),
            pl.BlockSpec((1, _N, _L), lambda c, rb: (c, 0, 0)),
            pl.BlockSpec((1, _L, _N), lambda c, rb: (c, 0, 0)),
            pl.BlockSpec((1, _R, _L), lambda c, rb: (c + 4, rb, 0)),
            pl.BlockSpec((1, _N, _L), lambda c, rb: (c + 4, 0, 0)),
            pl.BlockSpec((1, _L, _N), lambda c, rb: (c + 4, 0, 0)),
        ],
        out_specs=pl.BlockSpec((1, 1, _L), lambda c, rb: (c, 0, 0)),
        out_shape=jax.ShapeDtypeStruct((4, 1, _L), jnp.float32),
        scratch_shapes=[
            pltpu.VMEM((_N, _L), jnp.bfloat16),
            pltpu.VMEM((_N, _L), jnp.bfloat16),
            pltpu.VMEM((_N, _L), jnp.bfloat16),
            pltpu.VMEM((_N, _L), jnp.bfloat16),
            pltpu.SMEM((1, 1), jnp.float32),
        ],
        compiler_params=pltpu.CompilerParams(
            dimension_semantics=("parallel", "arbitrary")),
        interpret=_INTERPRET,
    )(ppad, ppad, ppadT, ppad, ppad, ppadT)
    return jnp.sum(partials[:, 0, 0]) * (1.0 / (4.0 * _N))


# speedup vs baseline: 229.5636x; 1.8205x over previous
import jax
import jax.numpy as jnp
from jax.experimental import pallas as pl
from jax.experimental.pallas import tpu as pltpu

_K = 10
_N = 2048
_R = 256
_NB = _N // _R
_L = 128
_INTERPRET = False


def _build_feats(pall, fhi_ref, flo_ref):
    x = pall[:, 0:1]
    y = pall[:, 1:2]
    z = pall[:, 2:3]
    f = jnp.concatenate(
        [x, y, z, x * x, y * y, z * z, x * y, x * z, y * z,
         jnp.ones((_N, 1), jnp.float32),
         jnp.zeros((_N, _L - 10), jnp.float32)], axis=1)
    hi = f.astype(jnp.bfloat16)
    fhi_ref[...] = hi
    flo_ref[...] = (f - hi.astype(jnp.float32)).astype(jnp.bfloat16)


def _normals(prow, pallT, fhi_ref, flo_ref):
    g = jax.lax.dot_general(prow, pallT, (((1,), (0,)), ((), ())),
                            preferred_element_type=jnp.float32)
    sq_row = jnp.sum(prow * prow, axis=1, keepdims=True)
    sq_all = jnp.sum(pallT * pallT, axis=0, keepdims=True)
    d = sq_row - 2.0 * g + sq_all

    t = jnp.min(d, axis=1, keepdims=True)
    for _ in range(_K - 1):
        t = jnp.min(jnp.where(d > t, d, jnp.inf), axis=1, keepdims=True)
    mask = (d <= t).astype(jnp.bfloat16)

    dims = (((1,), (0,)), ((), ()))
    s = (jax.lax.dot_general(mask, fhi_ref[...], dims,
                             preferred_element_type=jnp.float32)
         + jax.lax.dot_general(mask, flo_ref[...], dims,
                               preferred_element_type=jnp.float32))

    st = jnp.transpose(s)
    inv_c = 1.0 / st[9:10, :]
    mx = st[0:1, :] * inv_c
    my = st[1:2, :] * inv_c
    mz = st[2:3, :] * inv_c
    cxx = st[3:4, :] * inv_c - mx * mx
    cyy = st[4:5, :] * inv_c - my * my
    czz = st[5:6, :] * inv_c - mz * mz
    cxy = st[6:7, :] * inv_c - mx * my
    cxz = st[7:8, :] * inv_c - mx * mz
    cyz = st[8:9, :] * inv_c - my * mz

    q = (cxx + cyy + czz) * (1.0 / 3.0)
    aa = cxx - q
    bb = cyy - q
    cc = czz - q
    p2 = aa * aa + bb * bb + cc * cc + 2.0 * (cxy * cxy + cxz * cxz + cyz * cyz)
    p = jnp.sqrt(p2 * (1.0 / 6.0) + 1e-38)
    pinv = 1.0 / p
    b11 = aa * pinv
    b22 = bb * pinv
    b33 = cc * pinv
    b12 = cxy * pinv
    b13 = cxz * pinv
    b23 = cyz * pinv
    detb = (b11 * (b22 * b33 - b23 * b23)
            - b12 * (b12 * b33 - b23 * b13)
            + b13 * (b12 * b23 - b22 * b13))
    r = jnp.clip(detb * 0.5, -1.0, 1.0)
    lam = jnp.full_like(r, -2.0)
    for _ in range(12):
        f = lam * lam * lam - 3.0 * lam - 2.0 * r
        fp = 3.0 * lam * lam - 3.0 + 1e-10
        lam = lam - f / fp
    lmin = q + p * lam

    m11 = cxx - lmin
    m22 = cyy - lmin
    m33 = czz - lmin
    c12x = cxy * cyz - cxz * m22
    c12y = cxz * cxy - m11 * cyz
    c12z = m11 * m22 - cxy * cxy
    c13x = cxy * m33 - cxz * cyz
    c13y = cxz * cxz - m11 * m33
    c13z = m11 * cyz - cxy * cxz
    c23x = m22 * m33 - cyz * cyz
    c23y = cyz * cxz - cxy * m33
    c23z = cxy * cyz - m22 * cxz
    n12 = c12x * c12x + c12y * c12y + c12z * c12z
    n13 = c13x * c13x + c13y * c13y + c13z * c13z
    n23 = c23x * c23x + c23y * c23y + c23z * c23z
    use12 = jnp.logical_and(n12 >= n13, n12 >= n23)
    use13 = jnp.logical_and(jnp.logical_not(use12), n13 >= n23)
    vx = jnp.where(use12, c12x, jnp.where(use13, c13x, c23x))
    vy = jnp.where(use12, c12y, jnp.where(use13, c13y, c23y))
    vz = jnp.where(use12, c12z, jnp.where(use13, c13z, c23z))
    inv = jax.lax.rsqrt(vx * vx + vy * vy + vz * vz + 1e-38)
    return vx * inv, vy * inv, vz * inv


def _body(prow_p_ref, pall_p_ref, pallT_p_ref,
          prow_g_ref, pall_g_ref, pallT_g_ref,
          out_ref, fphi, fplo, fghi, fglo, acc):
    rb = pl.program_id(1)

    @pl.when(rb == 0)
    def _():
        _build_feats(pall_p_ref[0], fphi, fplo)
        _build_feats(pall_g_ref[0], fghi, fglo)
        acc[0, 0] = 0.0

    pnx, pny, pnz = _normals(prow_p_ref[0], pallT_p_ref[0], fphi, fplo)
    gnx, gny, gnz = _normals(prow_g_ref[0], pallT_g_ref[0], fghi, fglo)
    cos = pnx * gnx + pny * gny + pnz * gnz
    acc[0, 0] += jnp.sum(1.0 - jnp.abs(cos))

    @pl.when(rb == _NB - 1)
    def _():
        out_ref[...] = jnp.full((1, 1, _L), acc[0, 0], jnp.float32)


def kernel(pred, gt):
    pts = jnp.concatenate([pred, gt], axis=0)
    p = jnp.transpose(pts, (0, 2, 1))
    ppad = jnp.pad(p, ((0, 0), (0, 0), (0, _L - 3)))
    ppadT = jnp.transpose(ppad, (0, 2, 1))

    partials = pl.pallas_call(
        _body,
        grid=(4, _NB),
        in_specs=[
            pl.BlockSpec((1, _R, _L), lambda c, rb: (c, rb, 0)),
            pl.BlockSpec((1, _N, _L), lambda c, rb: (c, 0, 0)),
            pl.BlockSpec((1, _L, _N), lambda c, rb: (c, 0, 0)),
            pl.BlockSpec((1, _R, _L), lambda c, rb: (c + 4, rb, 0)),
            pl.BlockSpec((1, _N, _L), lambda c, rb: (c + 4, 0, 0)),
            pl.BlockSpec((1, _L, _N), lambda c, rb: (c + 4, 0, 0)),
        ],
        out_specs=pl.BlockSpec((1, 1, _L), lambda c, rb: (c, 0, 0)),
        out_shape=jax.ShapeDtypeStruct((4, 1, _L), jnp.float32),
        scratch_shapes=[
            pltpu.VMEM((_N, _L), jnp.bfloat16),
            pltpu.VMEM((_N, _L), jnp.bfloat16),
            pltpu.VMEM((_N, _L), jnp.bfloat16),
            pltpu.VMEM((_N, _L), jnp.bfloat16),
            pltpu.SMEM((1, 1), jnp.float32),
        ],
        compiler_params=pltpu.CompilerParams(
            dimension_semantics=("parallel", "arbitrary")),
        interpret=_INTERPRET,
    )(ppad, ppad, ppadT, ppad, ppad, ppadT)
    return jnp.sum(partials[:, 0, 0]) * (1.0 / (4.0 * _N))
